# baseline (speedup 1.0000x reference)
import functools

import jax
import jax.numpy as jnp
from jax import lax
from jax.experimental import pallas as pl
from jax.experimental.pallas import tpu as pltpu
from jax.experimental.pallas import tpu_sc as plsc

V = 100000
HALF_V = V // 2
D = 64
K = 512

_NC = 2
_NS = 16
_NW = _NC * _NS

_B = 4096 * 50
_BPW = _B // _NW
_SEQ = 50
_BATCH_PW = 4096 // _NW

_MM_BLOCK = 2000


def _table_matmul_body(code_lo_ref, code_hi_ref, w_ref, out_ref):
    xl = code_lo_ref[...].reshape(_MM_BLOCK, K)
    xh = code_hi_ref[...].reshape(_MM_BLOCK, K)
    out_ref[:, :D] = jnp.dot(xl, w_ref[...],
                             preferred_element_type=jnp.float32)
    out_ref[:, D:] = jnp.dot(xh, w_ref[...],
                             preferred_element_type=jnp.float32)


def _build_table(code4, w):
    grid = HALF_V // _MM_BLOCK
    rb = 4 * _MM_BLOCK
    return pl.pallas_call(
        _table_matmul_body,
        grid=(grid,),
        in_specs=[
            pl.BlockSpec((rb, 128), lambda i: (i, 0)),
            pl.BlockSpec((rb, 128), lambda i, g=grid: (i + g, 0)),
            pl.BlockSpec((K, D), lambda i: (0, 0)),
        ],
        out_specs=pl.BlockSpec((_MM_BLOCK, 2 * D), lambda i: (i, 0)),
        out_shape=jax.ShapeDtypeStruct((HALF_V, 2 * D), jnp.float32),
    )(code4, code4, w)


_GRP = 4
_NGRP = _BATCH_PW // _GRP
_RING = 3


def _gather_body(table_hbm, gidx_hbm, out_hbm, gidx_v, wbuf,
                 g0, g1, g2, o0, o1, o2):
    wid = lax.axis_index("s") * _NC + lax.axis_index("c")
    bbase = wid * _BATCH_PW
    pltpu.sync_copy(gidx_hbm.at[pl.ds(bbase, _BATCH_PW)], gidx_v)

    gsems = (g0, g1, g2)
    osems = (o0, o1, o2)
    gcopies = [None] * _RING
    ocopies = [None] * _RING
    for g in range(_NGRP + 1):
        s = g % _RING
        if g < _NGRP:
            if g >= _RING:
                ocopies[s].wait()
            gcopies[s] = [
                pltpu.async_copy(table_hbm.at[gidx_v.at[g * _GRP + q]],
                                 wbuf.at[s].at[q], gsems[s])
                for q in range(_GRP)
            ]
        if g >= 1:
            ps = (g - 1) % _RING
            for c in gcopies[ps]:
                c.wait()
            ocopies[ps] = pltpu.async_copy(
                wbuf.at[ps], out_hbm.at[pl.ds(bbase + (g - 1) * _GRP, _GRP)],
                osems[ps])
    for g in range(_NGRP - _RING, _NGRP):
        ocopies[g % _RING].wait()


def _gather(table64, gidx2d):
    mesh = plsc.VectorSubcoreMesh(core_axis_name="c", subcore_axis_name="s")
    return pl.kernel(
        _gather_body,
        out_type=jax.ShapeDtypeStruct((4096, _SEQ, D), jnp.float32),
        mesh=mesh,
        scratch_types=[
            pltpu.VMEM((_BATCH_PW, _SEQ), jnp.int32),
            pltpu.VMEM((_RING, _GRP, _SEQ, D), jnp.float32),
            pltpu.SemaphoreType.DMA,
            pltpu.SemaphoreType.DMA,
            pltpu.SemaphoreType.DMA,
            pltpu.SemaphoreType.DMA,
            pltpu.SemaphoreType.DMA,
            pltpu.SemaphoreType.DMA,
        ],
        compiler_params=pltpu.CompilerParams(use_tc_tiling_on_sc=False),
    )(table64, gidx2d)


@jax.jit
def kernel(input, code, codebook):
    code4 = code.reshape(4 * V, 128)
    w = codebook.reshape(K, D)
    table = _build_table(code4, w)
    table64 = table.reshape(V, D)
    v = input.astype(jnp.int32)
    hi = (v >= HALF_V).astype(jnp.int32)
    gidx2d = 2 * v - (2 * HALF_V - 1) * hi
    return _gather(table64, gidx2d)


# speedup vs baseline: 2.6851x; 2.6851x over previous
import functools

import jax
import jax.numpy as jnp
from jax import lax
from jax.experimental import pallas as pl
from jax.experimental.pallas import tpu as pltpu
from jax.experimental.pallas import tpu_sc as plsc

V = 100000
HALF_V = V // 2
D = 64
K = 512

_NC = 2
_NS = 16
_NW = _NC * _NS

_B = 4096 * 50
_BPW = _B // _NW
_SEQ = 50
_BATCH_PW = 4096 // _NW

_MM_BLOCK = 2000


def _table_matmul_body(code_lo_ref, code_hi_ref, w_ref, out_ref):
    out_ref[:, :D] = jnp.dot(code_lo_ref[...], w_ref[...],
                             preferred_element_type=jnp.float32)
    out_ref[:, D:] = jnp.dot(code_hi_ref[...], w_ref[...],
                             preferred_element_type=jnp.float32)


def _build_table(code2d, w):
    grid = HALF_V // _MM_BLOCK
    return pl.pallas_call(
        _table_matmul_body,
        grid=(grid,),
        in_specs=[
            pl.BlockSpec((_MM_BLOCK, K), lambda i: (i, 0)),
            pl.BlockSpec((_MM_BLOCK, K), lambda i, g=grid: (i + g, 0)),
            pl.BlockSpec((K, D), lambda i: (0, 0)),
        ],
        out_specs=pl.BlockSpec((_MM_BLOCK, 2 * D), lambda i: (i, 0)),
        out_shape=jax.ShapeDtypeStruct((HALF_V, 2 * D), jnp.float32),
    )(code2d, code2d, w)


_GRP = 4
_NGRP = _BATCH_PW // _GRP
_RING = 3


def _gather_body(table_hbm, gidx_hbm, out_hbm, gidx_v, wbuf,
                 g0, g1, g2, o0, o1, o2):
    wid = lax.axis_index("s") * _NC + lax.axis_index("c")
    bbase = wid * _BATCH_PW
    pltpu.sync_copy(gidx_hbm.at[pl.ds(bbase, _BATCH_PW)], gidx_v)

    gsems = (g0, g1, g2)
    osems = (o0, o1, o2)
    gcopies = [None] * _RING
    ocopies = [None] * _RING
    for g in range(_NGRP + 1):
        s = g % _RING
        if g < _NGRP:
            if g >= _RING:
                ocopies[s].wait()
            gcopies[s] = [
                pltpu.async_copy(table_hbm.at[gidx_v.at[g * _GRP + q]],
                                 wbuf.at[s].at[q], gsems[s])
                for q in range(_GRP)
            ]
        if g >= 1:
            ps = (g - 1) % _RING
            for c in gcopies[ps]:
                c.wait()
            ocopies[ps] = pltpu.async_copy(
                wbuf.at[ps], out_hbm.at[pl.ds(bbase + (g - 1) * _GRP, _GRP)],
                osems[ps])
    for g in range(_NGRP - _RING, _NGRP):
        ocopies[g % _RING].wait()


def _gather(table64, gidx2d):
    mesh = plsc.VectorSubcoreMesh(core_axis_name="c", subcore_axis_name="s")
    return pl.kernel(
        _gather_body,
        out_type=jax.ShapeDtypeStruct((4096, _SEQ, D), jnp.float32),
        mesh=mesh,
        scratch_types=[
            pltpu.VMEM((_BATCH_PW, _SEQ), jnp.int32),
            pltpu.VMEM((_RING, _GRP, _SEQ, D), jnp.float32),
            pltpu.SemaphoreType.DMA,
            pltpu.SemaphoreType.DMA,
            pltpu.SemaphoreType.DMA,
            pltpu.SemaphoreType.DMA,
            pltpu.SemaphoreType.DMA,
            pltpu.SemaphoreType.DMA,
        ],
        compiler_params=pltpu.CompilerParams(use_tc_tiling_on_sc=False),
    )(table64, gidx2d)


@jax.jit
def kernel(input, code, codebook):
    code2d = code.reshape(V, K).astype(jnp.bfloat16)
    w = codebook.reshape(K, D).astype(jnp.bfloat16)
    table = _build_table(code2d, w)
    table64 = table.reshape(V, D)
    v = input.astype(jnp.int32)
    hi = (v >= HALF_V).astype(jnp.int32)
    gidx2d = 2 * v - (2 * HALF_V - 1) * hi
    return _gather(table64, gidx2d)


# speedup vs baseline: 2.8503x; 1.0615x over previous
import functools

import jax
import jax.numpy as jnp
from jax import lax
from jax.experimental import pallas as pl
from jax.experimental.pallas import tpu as pltpu
from jax.experimental.pallas import tpu_sc as plsc

V = 100000
HALF_V = V // 2
D = 64
K = 512

_NC = 2
_NS = 16
_NW = _NC * _NS

_B = 4096 * 50
_BPW = _B // _NW
_SEQ = 50
_BATCH_PW = 4096 // _NW

_MM_BLOCK = 2000


def _table_matmul_body(code_lo_ref, code_hi_ref, w_ref, out_ref):
    out_ref[:, :D] = jnp.dot(code_lo_ref[...], w_ref[...],
                             preferred_element_type=jnp.float32)
    out_ref[:, D:] = jnp.dot(code_hi_ref[...], w_ref[...],
                             preferred_element_type=jnp.float32)


def _build_table(code2d, w):
    grid = HALF_V // _MM_BLOCK
    return pl.pallas_call(
        _table_matmul_body,
        grid=(grid,),
        in_specs=[
            pl.BlockSpec((_MM_BLOCK, K), lambda i: (i, 0)),
            pl.BlockSpec((_MM_BLOCK, K), lambda i, g=grid: (i + g, 0)),
            pl.BlockSpec((K, D), lambda i: (0, 0)),
        ],
        out_specs=pl.BlockSpec((_MM_BLOCK, 2 * D), lambda i: (i, 0)),
        out_shape=jax.ShapeDtypeStruct((HALF_V, 2 * D), jnp.float32),
    )(code2d, code2d, w)


_GRP = 4
_NGRP = _BATCH_PW // _GRP
_RING = 3


def _gather_body(table_hbm, gidx_hbm, out_hbm, gidx_v, wbuf,
                 g0, g1, g2, o0, o1, o2):
    wid = lax.axis_index("s") * _NC + lax.axis_index("c")
    bbase = wid * _BATCH_PW
    pltpu.sync_copy(gidx_hbm.at[pl.ds(bbase, _BATCH_PW)], gidx_v)

    gsems = (g0, g1, g2)
    osems = (o0, o1, o2)
    gcopies = [None] * _RING
    ocopies = [None] * _RING
    for g in range(_NGRP + 1):
        s = g % _RING
        if g < _NGRP:
            if g >= _RING:
                ocopies[s].wait()
            gcopies[s] = [
                pltpu.async_copy(table_hbm.at[gidx_v.at[g * _GRP + q]],
                                 wbuf.at[s].at[q], gsems[s])
                for q in range(_GRP)
            ]
        if g >= 1:
            ps = (g - 1) % _RING
            for c in gcopies[ps]:
                c.wait()
            ocopies[ps] = pltpu.async_copy(
                wbuf.at[ps], out_hbm.at[pl.ds(bbase + (g - 1) * _GRP, _GRP)],
                osems[ps])
    for g in range(_NGRP - _RING, _NGRP):
        ocopies[g % _RING].wait()


def _gather(table64, gidx2d):
    mesh = plsc.VectorSubcoreMesh(core_axis_name="c", subcore_axis_name="s")
    return pl.kernel(
        _gather_body,
        out_type=jax.ShapeDtypeStruct((4096, _SEQ, D), jnp.float32),
        mesh=mesh,
        scratch_types=[
            pltpu.VMEM((_BATCH_PW, _SEQ), jnp.int32),
            pltpu.VMEM((_RING, _GRP, _SEQ, D), jnp.float32),
            pltpu.SemaphoreType.DMA,
            pltpu.SemaphoreType.DMA,
            pltpu.SemaphoreType.DMA,
            pltpu.SemaphoreType.DMA,
            pltpu.SemaphoreType.DMA,
            pltpu.SemaphoreType.DMA,
        ],
        compiler_params=pltpu.CompilerParams(use_tc_tiling_on_sc=False),
    )(table64, gidx2d)


@jax.jit
def kernel(input, code, codebook):
    code2d = code.reshape(V, K)
    w = codebook.reshape(K, D)
    table = _build_table(code2d, w)
    table64 = table.reshape(V, D)
    v = input.astype(jnp.int32)
    hi = (v >= HALF_V).astype(jnp.int32)
    gidx2d = 2 * v - (2 * HALF_V - 1) * hi
    return _gather(table64, gidx2d)
